# baseline (speedup 1.0000x reference)
import jax
import jax.numpy as jnp
from jax import lax
from jax.experimental import pallas as pl
from jax.experimental.pallas import tpu as pltpu
from jax.experimental.pallas import tpu_sc as plsc

N = 10000
E = 320000
F = 128
KER = 4
SF = F * KER

NC = 2
NS = 16
L = 16
NW = NC * NS

C = 16
BLK = 8
CHUNKS_PER_TILE = 632
E_PAD = C * NW * CHUNKS_PER_TILE
N_PAD = 10240
ROWS_PER_TILE = N_PAD // NS
PAD_SRC = N_PAD - 8


def _matmul_body(x_ref, w_ref, o_ref):
    o_ref[...] = jnp.dot(x_ref[...], w_ref[...],
                         preferred_element_type=jnp.float32)


def _support_matmul(x, w_all):
    rows = 1000
    return pl.pallas_call(
        _matmul_body,
        grid=(N // rows,),
        in_specs=[
            pl.BlockSpec((rows, F), lambda i: (i, 0)),
            pl.BlockSpec((F, SF), lambda i: (0, 0)),
        ],
        out_specs=pl.BlockSpec((rows, SF), lambda i: (i, 0)),
        out_shape=jax.ShapeDtypeStruct((N, SF), jnp.float32),
    )(x, w_all)


def _combine_body(p_ref, b_ref, o_ref):
    o_ref[...] = p_ref[0] + p_ref[1] + b_ref[...][None, :]


def _combine(parts, bias):
    rows = 1000
    return pl.pallas_call(
        _combine_body,
        grid=(N // rows,),
        in_specs=[
            pl.BlockSpec((NC, rows, F), lambda i: (0, i, 0)),
            pl.BlockSpec((F,), lambda i: (0,)),
        ],
        out_specs=pl.BlockSpec((rows, F), lambda i: (i, 0)),
        out_shape=jax.ShapeDtypeStruct((N, F), jnp.float32),
    )(parts, bias)


def _sc_body(s_hbm, edges_hbm, d0_hbm, d1_hbm, d2_hbm, params_hbm,
             zeros_hbm, out_hbm, acc_sh, params_v, sd_v, rows_v, dsrc_v,
             ddst_v, vbuf_v, msg_v, sem_r, sem_d1, sem_d2):
    cid = lax.axis_index("c")
    sid = lax.axis_index("s")
    wid = cid * NS + sid
    base = wid * CHUNKS_PER_TILE

    pltpu.sync_copy(params_hbm, params_v)
    pltpu.sync_copy(zeros_hbm,
                    acc_sh.at[pl.ds(sid * ROWS_PER_TILE, ROWS_PER_TILE)])
    plsc.subcore_barrier()

    pvec = params_v[...]
    mu = [[pvec[4 * k + j] for k in range(KER)] for j in range(3)]
    nhs = [pvec[4 * k + 3] for k in range(KER)]

    def sd_row(jn, which):
        return ((jn // BLK) % 2) * (2 * BLK) + (jn % BLK) * 2 + which

    def descs(jn, slot):
        ds_ = [
            pltpu.make_async_copy(s_hbm.at[sd_v.at[sd_row(jn, 1)]],
                                  rows_v.at[slot], sem_r),
        ]
        for d, col in enumerate((d0_hbm, d1_hbm, d2_hbm)):
            ds_.append(pltpu.make_async_copy(col.at[sd_v.at[sd_row(jn, 0)]],
                                             dsrc_v.at[slot * 3 + d],
                                             sem_d1))
            ds_.append(pltpu.make_async_copy(col.at[sd_v.at[sd_row(jn, 1)]],
                                             ddst_v.at[slot * 3 + d],
                                             sem_d2))
        return ds_

    def load_block(bk):
        pltpu.sync_copy(edges_hbm.at[base // BLK + bk],
                        sd_v.at[pl.ds((bk % 2) * (2 * BLK), 2 * BLK)])

    load_block(0)
    for g in descs(0, 0):
        g.start()
        g.wait()

    def process(j, slot):
        for g in range(C // L):
            a0 = (dsrc_v[slot * 3 + 0, pl.ds(g * L, L)]
                  - ddst_v[slot * 3 + 0, pl.ds(g * L, L)])
            a1 = (dsrc_v[slot * 3 + 1, pl.ds(g * L, L)]
                  - ddst_v[slot * 3 + 1, pl.ds(g * L, L)])
            a2 = (dsrc_v[slot * 3 + 2, pl.ds(g * L, L)]
                  - ddst_v[slot * 3 + 2, pl.ds(g * L, L)])
            for k in range(KER):
                dd0 = a0 - mu[0][k]
                dd1 = a1 - mu[1][k]
                dd2 = a2 - mu[2][k]
                ssq = dd0 * dd0 + dd1 * dd1 + dd2 * dd2
                vbuf_v[k, pl.ds(g * L, L)] = jnp.exp(nhs[k] * ssq)
        for g in range(C // L):
            vk = [vbuf_v[k, pl.ds(g * L, L)] for k in range(KER)]
            for i in range(L):
                e = g * L + i
                for f in range(F // L):
                    m = (vk[0][i] * rows_v[slot, e, pl.ds(0 * F + f * L, L)]
                         + vk[1][i] * rows_v[slot, e, pl.ds(1 * F + f * L, L)]
                         + vk[2][i] * rows_v[slot, e, pl.ds(2 * F + f * L, L)]
                         + vk[3][i] * rows_v[slot, e, pl.ds(3 * F + f * L, L)])
                    msg_v[e, pl.ds(f * L, L)] = m
        pltpu.sync_copy(msg_v, acc_sh.at[sd_v.at[sd_row(j, 0)]], add=True)

    def pair_body(p, carry):
        j0 = 2 * p
        j1 = j0 + 1
        j2 = j0 + 2
        for g in descs(j1, 1):
            g.start()
        process(j0, 0)
        for g in descs(j1, 1):
            g.wait()

        @pl.when(jnp.logical_and(j2 % BLK == 0, j2 < CHUNKS_PER_TILE))
        def _():
            load_block(j2 // BLK)

        do_next = j2 < CHUNKS_PER_TILE

        @pl.when(do_next)
        def _():
            for g in descs(j2, 0):
                g.start()

        process(j1, 1)

        @pl.when(do_next)
        def _():
            for g in descs(j2, 0):
                g.wait()

        return carry

    lax.fori_loop(0, CHUNKS_PER_TILE // 2, pair_body, 0)
    plsc.subcore_barrier()
    pltpu.sync_copy(acc_sh.at[pl.ds(sid * ROWS_PER_TILE, ROWS_PER_TILE)],
                    out_hbm.at[cid, pl.ds(sid * ROWS_PER_TILE, ROWS_PER_TILE)])


_sc_aggregate = pl.kernel(
    _sc_body,
    out_type=jax.ShapeDtypeStruct((NC, N_PAD, F), jnp.float32),
    mesh=plsc.VectorSubcoreMesh(core_axis_name="c", subcore_axis_name="s",
                                num_cores=NC, num_subcores=NS),
    scratch_types=[
        pltpu.VMEM_SHARED((N_PAD, F), jnp.float32),
        pltpu.VMEM((4 * KER,), jnp.float32),
        pltpu.VMEM((4 * BLK, C), jnp.int32),
        pltpu.VMEM((2, C, SF), jnp.float32),
        pltpu.VMEM((6, C), jnp.float32),
        pltpu.VMEM((6, C), jnp.float32),
        pltpu.VMEM((KER, C), jnp.float32),
        pltpu.VMEM((C, F), jnp.float32),
        pltpu.SemaphoreType.DMA,
        pltpu.SemaphoreType.DMA,
        pltpu.SemaphoreType.DMA,
    ],
    compiler_params=pltpu.CompilerParams(needs_layout_passes=False),
)


def kernel(x, edge_index, weight, bias, mu, sig):
    w_all = weight.transpose(0, 2, 1).reshape(F, SF)
    s = _support_matmul(x, w_all)
    pad = jnp.full((2, E_PAD - E), 0, jnp.int32).at[0, :].set(PAD_SRC)
    edges = (jnp.concatenate([edge_index, pad], axis=1)
             .reshape(2, E_PAD // (BLK * C), BLK, C).transpose(1, 2, 0, 3)
             .reshape(E_PAD // (BLK * C), 2 * BLK, C))
    dcols = jnp.zeros((3, N_PAD), jnp.float32).at[:, :N].set(x[:, :3].T)
    params = jnp.concatenate([mu, -0.5 * sig], axis=0).T.reshape(-1)
    zeros = jnp.zeros((ROWS_PER_TILE, F), jnp.float32)
    parts = _sc_aggregate(s, edges, dcols[0], dcols[1], dcols[2], params,
                          zeros)
    return _combine(parts, bias)


# speedup vs baseline: 1.0154x; 1.0154x over previous
import jax
import jax.numpy as jnp
from jax import lax
from jax.experimental import pallas as pl
from jax.experimental.pallas import tpu as pltpu
from jax.experimental.pallas import tpu_sc as plsc

N = 10000
E = 320000
F = 128
KER = 4
SF = F * KER

NC = 2
NS = 16
L = 16
NW = NC * NS

C = 16
BLK = 8
CHUNKS_PER_TILE = 632
E_PAD = C * NW * CHUNKS_PER_TILE
N_PAD = 10240
ROWS_PER_TILE = N_PAD // NS
PAD_SRC = N_PAD - 8


def _matmul_body(x_ref, w_ref, o_ref):
    o_ref[...] = jnp.dot(x_ref[...], w_ref[...],
                         preferred_element_type=jnp.float32)


def _support_matmul(x, w_all):
    rows = 1000
    return pl.pallas_call(
        _matmul_body,
        grid=(N // rows,),
        in_specs=[
            pl.BlockSpec((rows, F), lambda i: (i, 0)),
            pl.BlockSpec((F, SF), lambda i: (0, 0)),
        ],
        out_specs=pl.BlockSpec((rows, SF), lambda i: (i, 0)),
        out_shape=jax.ShapeDtypeStruct((N, SF), jnp.float32),
    )(x, w_all)


def _combine_body(p_ref, b_ref, o_ref):
    o_ref[...] = p_ref[0] + p_ref[1] + b_ref[...][None, :]


def _combine(parts, bias):
    rows = 1000
    return pl.pallas_call(
        _combine_body,
        grid=(N // rows,),
        in_specs=[
            pl.BlockSpec((NC, rows, F), lambda i: (0, i, 0)),
            pl.BlockSpec((F,), lambda i: (0,)),
        ],
        out_specs=pl.BlockSpec((rows, F), lambda i: (i, 0)),
        out_shape=jax.ShapeDtypeStruct((N, F), jnp.float32),
    )(parts, bias)


def _sc_body(s_hbm, edges_hbm, d0_hbm, d1_hbm, d2_hbm, params_hbm,
             zeros_hbm, out_hbm, acc_sh, params_v, sd_v, rows_v, dsrc_v,
             ddst_v, vbuf_v, sidx_v, msg_v, sem_r, sem_d1, sem_d2):
    cid = lax.axis_index("c")
    sid = lax.axis_index("s")
    wid = cid * NS + sid
    base = wid * CHUNKS_PER_TILE

    pltpu.sync_copy(params_hbm, params_v)
    pltpu.sync_copy(zeros_hbm,
                    acc_sh.at[pl.ds(sid * ROWS_PER_TILE, ROWS_PER_TILE)])
    plsc.subcore_barrier()

    pvec = params_v[...]
    mu = [[pvec[4 * k + j] for k in range(KER)] for j in range(3)]
    nhs = [pvec[4 * k + 3] for k in range(KER)]

    BW = BLK * C

    def descs(jn, slot):
        bsl = ((jn // BLK) % 2) * 2
        return [
            pltpu.make_async_copy(
                s_hbm.at[sd_v.at[bsl + 1, pl.ds((jn % BLK) * C, C)]],
                rows_v.at[slot], sem_r),
        ]

    def ddescs(bk):
        bsl = (bk % 2) * 2
        ds_ = []
        for d, col in enumerate((d0_hbm, d1_hbm, d2_hbm)):
            ds_.append(pltpu.make_async_copy(col.at[sd_v.at[bsl + 0]],
                                             dsrc_v.at[(bk % 2) * 3 + d],
                                             sem_d1))
            ds_.append(pltpu.make_async_copy(col.at[sd_v.at[bsl + 1]],
                                             ddst_v.at[(bk % 2) * 3 + d],
                                             sem_d2))
        return ds_

    def load_block(bk):
        pltpu.sync_copy(edges_hbm.at[base // BLK + bk],
                        sd_v.at[pl.ds((bk % 2) * 2, 2)])

    def vcompute(bk):
        bsl = (bk % 2) * 3
        for g in range(BW // L):
            a0 = (dsrc_v[bsl + 0, pl.ds(g * L, L)]
                  - ddst_v[bsl + 0, pl.ds(g * L, L)])
            a1 = (dsrc_v[bsl + 1, pl.ds(g * L, L)]
                  - ddst_v[bsl + 1, pl.ds(g * L, L)])
            a2 = (dsrc_v[bsl + 2, pl.ds(g * L, L)]
                  - ddst_v[bsl + 2, pl.ds(g * L, L)])
            for k in range(KER):
                dd0 = a0 - mu[0][k]
                dd1 = a1 - mu[1][k]
                dd2 = a2 - mu[2][k]
                ssq = dd0 * dd0 + dd1 * dd1 + dd2 * dd2
                vbuf_v[(bk % 2) * KER + k, pl.ds(g * L, L)] = (
                    jnp.exp(nhs[k] * ssq))

    load_block(0)
    for g in ddescs(0):
        g.start()
    for g in ddescs(0):
        g.wait()
    vcompute(0)
    for g in descs(0, 0):
        g.start()
        g.wait()

    def process(j, slot):
        bpar = (j // BLK) % 2
        boff = (j % BLK) * C
        sidx_v[slot, :] = sd_v[bpar * 2, pl.ds(boff, C)]
        vk = [vbuf_v[bpar * KER + k, pl.ds(boff, C)] for k in range(KER)]
        for e in range(C):
            for f in range(F // L):
                m = (vk[0][e] * rows_v[slot, e, pl.ds(0 * F + f * L, L)]
                     + vk[1][e] * rows_v[slot, e, pl.ds(1 * F + f * L, L)]
                     + vk[2][e] * rows_v[slot, e, pl.ds(2 * F + f * L, L)]
                     + vk[3][e] * rows_v[slot, e, pl.ds(3 * F + f * L, L)])
                msg_v[e, pl.ds(f * L, L)] = m
        pltpu.sync_copy(msg_v, acc_sh.at[sidx_v.at[slot]], add=True)

    def pair_body(p, carry):
        j0 = 2 * p
        j1 = j0 + 1
        j2 = j0 + 2
        for g in descs(j1, 1):
            g.start()
        process(j0, 0)
        for g in descs(j1, 1):
            g.wait()

        new_block = jnp.logical_and(j2 % BLK == 0, j2 < CHUNKS_PER_TILE)

        @pl.when(new_block)
        def _():
            load_block(j2 // BLK)
            for g in ddescs(j2 // BLK):
                g.start()

        do_next = j2 < CHUNKS_PER_TILE

        @pl.when(do_next)
        def _():
            for g in descs(j2, 0):
                g.start()

        process(j1, 1)

        @pl.when(new_block)
        def _():
            for g in ddescs(j2 // BLK):
                g.wait()
            vcompute(j2 // BLK)

        @pl.when(do_next)
        def _():
            for g in descs(j2, 0):
                g.wait()

        return carry

    lax.fori_loop(0, CHUNKS_PER_TILE // 2, pair_body, 0)
    plsc.subcore_barrier()
    pltpu.sync_copy(acc_sh.at[pl.ds(sid * ROWS_PER_TILE, ROWS_PER_TILE)],
                    out_hbm.at[cid, pl.ds(sid * ROWS_PER_TILE, ROWS_PER_TILE)])


_sc_aggregate = pl.kernel(
    _sc_body,
    out_type=jax.ShapeDtypeStruct((NC, N_PAD, F), jnp.float32),
    mesh=plsc.VectorSubcoreMesh(core_axis_name="c", subcore_axis_name="s",
                                num_cores=NC, num_subcores=NS),
    scratch_types=[
        pltpu.VMEM_SHARED((N_PAD, F), jnp.float32),
        pltpu.VMEM((4 * KER,), jnp.float32),
        pltpu.VMEM((4, BLK * C), jnp.int32),
        pltpu.VMEM((2, C, SF), jnp.float32),
        pltpu.VMEM((6, BLK * C), jnp.float32),
        pltpu.VMEM((6, BLK * C), jnp.float32),
        pltpu.VMEM((2 * KER, BLK * C), jnp.float32),
        pltpu.VMEM((2, C), jnp.int32),
        pltpu.VMEM((C, F), jnp.float32),
        pltpu.SemaphoreType.DMA,
        pltpu.SemaphoreType.DMA,
        pltpu.SemaphoreType.DMA,
    ],
    compiler_params=pltpu.CompilerParams(needs_layout_passes=False),
)


def kernel(x, edge_index, weight, bias, mu, sig):
    w_all = weight.transpose(0, 2, 1).reshape(F, SF)
    s = _support_matmul(x, w_all)
    pad = jnp.full((2, E_PAD - E), 0, jnp.int32).at[0, :].set(PAD_SRC)
    edges = (jnp.concatenate([edge_index, pad], axis=1)
             .reshape(2, E_PAD // (BLK * C), BLK * C).transpose(1, 0, 2))
    dcols = jnp.zeros((3, N_PAD), jnp.float32).at[:, :N].set(x[:, :3].T)
    params = jnp.concatenate([mu, -0.5 * sig], axis=0).T.reshape(-1)
    zeros = jnp.zeros((ROWS_PER_TILE, F), jnp.float32)
    parts = _sc_aggregate(s, edges, dcols[0], dcols[1], dcols[2], params,
                          zeros)
    return _combine(parts, bias)
